# Initial kernel scaffold; baseline (speedup 1.0000x reference)
import jax
import jax.numpy as jnp
from jax.experimental import pallas as pl


def kernel(x, table0, table1, table2, table3):
    raise NotImplementedError("write your pallas kernel here")


# speedup vs baseline: 14.4088x; 14.4088x over previous
import functools
import math

import jax
import jax.numpy as jnp
import numpy as np
from jax import lax
from jax.experimental import pallas as pl
from jax.experimental.pallas import tpu as pltpu
from jax.experimental.pallas import tpu_sc as plsc

_RESOLUTIONS = (16, 32, 64, 128)
_TABLE_SIZES = (512, 2048, 8196, 32768)
_P1 = np.int32(np.uint32(2654435761))
_P2 = np.int32(np.uint32(805459861))

_NW = 32
_LANES = 16


def _mod8196(h):
    low2 = jnp.bitwise_and(h, 3)
    h2 = lax.shift_right_logical(h, 2)
    a = lax.shift_right_logical(h2, 11)
    b = jnp.bitwise_and(h2, 2047)
    r1 = b - a + np.int32(256 * 2049)
    a2 = lax.shift_right_logical(r1, 11)
    b2 = jnp.bitwise_and(r1, 2047)
    r2 = b2 - a2 + np.int32(2049)
    t = r2 - np.int32(2049)
    r = t + jnp.bitwise_and(np.int32(2049), lax.shift_right_arithmetic(t, 31))
    return jnp.bitwise_or(lax.shift_left(r, 2), low2)


def _layout(n):
    if n >= _NW * 3136:
        chunk = 3136
    else:
        chunk = 32
    tile_n = int(math.ceil(n / (_NW * chunk))) * chunk
    return chunk, tile_n, tile_n // chunk, _NW * tile_n


def _body(x_hbm, t0l, t0h, t1l, t1h, t2l, t2h, t3l, t3h,
          out_hbm, mask_hbm,
          v0l, v0h, v1l, v1h, v2l, v2h, v3l, v3h,
          x_v, out_v, mask_v, *, chunk, chunks, tile_n):
    wid = lax.axis_index("s") * 2 + lax.axis_index("c")
    pltpu.sync_copy(t0l, v0l)
    pltpu.sync_copy(t0h, v0h)
    pltpu.sync_copy(t1l, v1l)
    pltpu.sync_copy(t1h, v1h)
    pltpu.sync_copy(t2l, v2l)
    pltpu.sync_copy(t2h, v2h)
    pltpu.sync_copy(t3l, v3l)
    pltpu.sync_copy(t3h, v3h)

    tbls = ((v0l, v0h), (v1l, v1h), (v2l, v2h), (v3l, v3h))
    iota = lax.iota(jnp.int32, _LANES)
    iota3 = iota * 3
    iota8 = iota * 8

    def group_body(g, _):
        lanebase = g * _LANES
        ix = iota3 + lanebase * 3
        x0 = plsc.load_gather(x_v, [ix])
        x1 = plsc.load_gather(x_v, [ix + 1])
        x2 = plsc.load_gather(x_v, [ix + 2])
        m = ((x0 >= 0.0) & (x0 <= 1.0) & (x1 >= 0.0) & (x1 <= 1.0)
             & (x2 >= 0.0) & (x2 <= 1.0))
        xc = (jnp.minimum(jnp.maximum(x0, 0.0), 1.0),
              jnp.minimum(jnp.maximum(x1, 0.0), 1.0),
              jnp.minimum(jnp.maximum(x2, 0.0), 1.0))
        for lvl in range(4):
            res = _RESOLUTIONS[lvl]
            tsize = _TABLE_SIZES[lvl]
            tlo, thi = tbls[lvl]
            scale = np.float32(res - 1)
            p0 = xc[0] * scale
            p1 = xc[1] * scale
            p2 = xc[2] * scale
            b0 = jnp.minimum(p0.astype(jnp.int32), np.int32(res - 2))
            b1 = jnp.minimum(p1.astype(jnp.int32), np.int32(res - 2))
            b2 = jnp.minimum(p2.astype(jnp.int32), np.int32(res - 2))
            f0 = p0 - b0.astype(jnp.float32)
            f1 = p1 - b1.astype(jnp.float32)
            f2 = p2 - b2.astype(jnp.float32)
            hx = (b0, b0 + 1)
            hy0 = b1 * _P1
            hy = (hy0, hy0 + _P1)
            hz0 = b2 * _P2
            hz = (hz0, hz0 + _P2)
            wx = (1.0 - f0, f0)
            wy = (1.0 - f1, f1)
            wz = (1.0 - f2, f2)
            acc0 = jnp.zeros((_LANES,), jnp.float32)
            acc1 = jnp.zeros((_LANES,), jnp.float32)
            for dx in (0, 1):
                for dy in (0, 1):
                    hxy = jnp.bitwise_xor(hx[dx], hy[dy])
                    wxy = wx[dx] * wy[dy]
                    for dz in (0, 1):
                        h = jnp.bitwise_xor(hxy, hz[dz])
                        if tsize == 8196:
                            idx = _mod8196(h)
                        else:
                            idx = jnp.bitwise_and(h, np.int32(tsize - 1))
                        w = wxy * wz[dz]
                        acc0 = acc0 + w * plsc.load_gather(tlo, [idx])
                        acc1 = acc1 + w * plsc.load_gather(thi, [idx])
            acc0 = jnp.where(m, acc0, 0.0)
            acc1 = jnp.where(m, acc1, 0.0)
            ox = iota8 + (lanebase * 8 + 2 * lvl)
            plsc.store_scatter(out_v, [ox], acc0)
            plsc.store_scatter(out_v, [ox + 1], acc1)
        mask_v[pl.ds(lanebase, _LANES)] = jnp.where(m, 1, 0).astype(jnp.int32)
        return 0

    def chunk_body(ci, _):
        row0 = wid * tile_n + ci * chunk
        pltpu.sync_copy(x_hbm.at[pl.ds(row0 * 3, chunk * 3)], x_v)
        lax.fori_loop(0, chunk // _LANES, group_body, 0, unroll=False)
        pltpu.sync_copy(out_v, out_hbm.at[pl.ds(row0 * 8, chunk * 8)])
        pltpu.sync_copy(mask_v, mask_hbm.at[pl.ds(row0, chunk)])
        return 0

    lax.fori_loop(0, chunks, chunk_body, 0, unroll=False)


def _build(n_pts, interpret=False):
    chunk, tile_n, chunks, n_pad = _layout(n_pts)
    mesh = plsc.VectorSubcoreMesh(core_axis_name="c", subcore_axis_name="s",
                                  num_cores=2, num_subcores=16)
    scratch = [pltpu.VMEM((ts,), jnp.float32)
               for ts in _TABLE_SIZES for _ in range(2)]
    scratch += [
        pltpu.VMEM((chunk * 3,), jnp.float32),
        pltpu.VMEM((chunk * 8,), jnp.float32),
        pltpu.VMEM((chunk,), jnp.int32),
    ]
    return pl.kernel(
        functools.partial(_body, chunk=chunk, chunks=chunks, tile_n=tile_n),
        out_type=[
            jax.ShapeDtypeStruct((n_pad * 8,), jnp.float32),
            jax.ShapeDtypeStruct((n_pad,), jnp.int32),
        ],
        mesh=mesh,
        scratch_types=scratch,
        compiler_params=pltpu.CompilerParams(needs_layout_passes=False),
        interpret=interpret,
    ), n_pad


@jax.jit
def kernel(x, table0, table1, table2, table3):
    n = x.shape[0]
    k, n_pad = _build(n)
    x_pad = jnp.pad(x, ((0, n_pad - n), (0, 0))).reshape(-1)
    planes = []
    for t in (table0, table1, table2, table3):
        planes.append(t[:, 0])
        planes.append(t[:, 1])
    feats, mask_i32 = k(x_pad, *planes)
    return feats.reshape(n_pad, 8)[:n], mask_i32[:n] != 0


# speedup vs baseline: 15.5988x; 1.0826x over previous
import functools
import math

import jax
import jax.numpy as jnp
import numpy as np
from jax import lax
from jax.experimental import pallas as pl
from jax.experimental.pallas import tpu as pltpu
from jax.experimental.pallas import tpu_sc as plsc

_RESOLUTIONS = (16, 32, 64, 128)
_TABLE_SIZES = (512, 2048, 8196, 32768)
_P1 = np.int32(np.uint32(2654435761))
_P2 = np.int32(np.uint32(805459861))

_NW = 32
_LANES = 16


def _mod8196(h):
    low2 = jnp.bitwise_and(h, 3)
    h2 = lax.shift_right_logical(h, 2)
    a = lax.shift_right_logical(h2, 11)
    b = jnp.bitwise_and(h2, 2047)
    r1 = b - a + np.int32(256 * 2049)
    a2 = lax.shift_right_logical(r1, 11)
    b2 = jnp.bitwise_and(r1, 2047)
    r2 = b2 - a2 + np.int32(2049)
    t = r2 - np.int32(2049)
    r = t + jnp.bitwise_and(np.int32(2049), lax.shift_right_arithmetic(t, 31))
    return jnp.bitwise_or(lax.shift_left(r, 2), low2)


def _layout(n):
    chunk = 2000 if n % 2000 == 0 else 16
    assert n % chunk == 0
    return chunk, n // chunk


def _body(x_hbm, t0l, t0h, t1l, t1h, t2l, t2h, t3l, t3h,
          out_hbm, mask_hbm,
          v0l, v0h, v1l, v1h, v2l, v2h, v3l, v3h,
          x_v, out_v, mask_v, *, chunk, nchunks):
    wid = lax.axis_index("s") * 2 + lax.axis_index("c")
    base_chunks = nchunks // _NW
    n_mine = base_chunks + jnp.where(wid < nchunks - base_chunks * _NW, 1, 0)
    pltpu.sync_copy(t0l, v0l)
    pltpu.sync_copy(t0h, v0h)
    pltpu.sync_copy(t1l, v1l)
    pltpu.sync_copy(t1h, v1h)
    pltpu.sync_copy(t2l, v2l)
    pltpu.sync_copy(t2h, v2h)
    pltpu.sync_copy(t3l, v3l)
    pltpu.sync_copy(t3h, v3h)

    tbls = ((v0l, v0h), (v1l, v1h), (v2l, v2h), (v3l, v3h))
    iota = lax.iota(jnp.int32, _LANES)
    iota3 = iota * 3
    iota8 = iota * 8

    def group_body(g, _):
        lanebase = g * _LANES
        ix = iota3 + lanebase * 3
        x0 = plsc.load_gather(x_v, [ix])
        x1 = plsc.load_gather(x_v, [ix + 1])
        x2 = plsc.load_gather(x_v, [ix + 2])
        m = ((x0 >= 0.0) & (x0 <= 1.0) & (x1 >= 0.0) & (x1 <= 1.0)
             & (x2 >= 0.0) & (x2 <= 1.0))
        xc = (jnp.minimum(jnp.maximum(x0, 0.0), 1.0),
              jnp.minimum(jnp.maximum(x1, 0.0), 1.0),
              jnp.minimum(jnp.maximum(x2, 0.0), 1.0))
        for lvl in range(4):
            res = _RESOLUTIONS[lvl]
            tsize = _TABLE_SIZES[lvl]
            tlo, thi = tbls[lvl]
            scale = np.float32(res - 1)
            p0 = xc[0] * scale
            p1 = xc[1] * scale
            p2 = xc[2] * scale
            b0 = jnp.minimum(p0.astype(jnp.int32), np.int32(res - 2))
            b1 = jnp.minimum(p1.astype(jnp.int32), np.int32(res - 2))
            b2 = jnp.minimum(p2.astype(jnp.int32), np.int32(res - 2))
            f0 = p0 - b0.astype(jnp.float32)
            f1 = p1 - b1.astype(jnp.float32)
            f2 = p2 - b2.astype(jnp.float32)
            hx = (b0, b0 + 1)
            hy0 = b1 * _P1
            hy = (hy0, hy0 + _P1)
            hz0 = b2 * _P2
            hz = (hz0, hz0 + _P2)
            wx = (1.0 - f0, f0)
            wy = (1.0 - f1, f1)
            wz = (1.0 - f2, f2)
            acc0 = jnp.zeros((_LANES,), jnp.float32)
            acc1 = jnp.zeros((_LANES,), jnp.float32)
            for dx in (0, 1):
                for dy in (0, 1):
                    hxy = jnp.bitwise_xor(hx[dx], hy[dy])
                    wxy = wx[dx] * wy[dy]
                    for dz in (0, 1):
                        h = jnp.bitwise_xor(hxy, hz[dz])
                        if tsize == 8196:
                            idx = _mod8196(h)
                        else:
                            idx = jnp.bitwise_and(h, np.int32(tsize - 1))
                        w = wxy * wz[dz]
                        acc0 = acc0 + w * plsc.load_gather(tlo, [idx])
                        acc1 = acc1 + w * plsc.load_gather(thi, [idx])
            acc0 = jnp.where(m, acc0, 0.0)
            acc1 = jnp.where(m, acc1, 0.0)
            ox = iota8 + (lanebase * 8 + 2 * lvl)
            plsc.store_scatter(out_v, [ox], acc0)
            plsc.store_scatter(out_v, [ox + 1], acc1)
        mask_v[pl.ds(lanebase, _LANES)] = jnp.where(m, 1, 0).astype(jnp.int32)
        return 0

    def chunk_body(ci, _):
        row0 = (ci * _NW + wid) * chunk
        pltpu.sync_copy(x_hbm.at[pl.ds(row0 * 3, chunk * 3)], x_v)
        lax.fori_loop(0, chunk // _LANES, group_body, 0, unroll=False)
        pltpu.sync_copy(out_v, out_hbm.at[pl.ds(row0 * 8, chunk * 8)])
        pltpu.sync_copy(mask_v, mask_hbm.at[pl.ds(row0, chunk)])
        return 0

    lax.fori_loop(0, n_mine, chunk_body, 0, unroll=False)


def _build(n_pts, interpret=False):
    chunk, nchunks = _layout(n_pts)
    mesh = plsc.VectorSubcoreMesh(core_axis_name="c", subcore_axis_name="s",
                                  num_cores=2, num_subcores=16)
    scratch = [pltpu.VMEM((ts,), jnp.float32)
               for ts in _TABLE_SIZES for _ in range(2)]
    scratch += [
        pltpu.VMEM((chunk * 3,), jnp.float32),
        pltpu.VMEM((chunk * 8,), jnp.float32),
        pltpu.VMEM((chunk,), jnp.int32),
    ]
    return pl.kernel(
        functools.partial(_body, chunk=chunk, nchunks=nchunks),
        out_type=[
            jax.ShapeDtypeStruct((n_pts * 8,), jnp.float32),
            jax.ShapeDtypeStruct((n_pts,), jnp.int32),
        ],
        mesh=mesh,
        scratch_types=scratch,
        compiler_params=pltpu.CompilerParams(needs_layout_passes=False),
        interpret=interpret,
    )


@jax.jit
def kernel(x, table0, table1, table2, table3):
    n = x.shape[0]
    k = _build(n)
    planes = []
    for t in (table0, table1, table2, table3):
        planes.append(t[:, 0])
        planes.append(t[:, 1])
    feats, mask_i32 = k(x.reshape(-1), *planes)
    return feats.reshape(n, 8), mask_i32 != 0


# speedup vs baseline: 39.5124x; 2.5330x over previous
import functools
import math

import jax
import jax.numpy as jnp
import numpy as np
from jax import lax
from jax.experimental import pallas as pl
from jax.experimental.pallas import tpu as pltpu
from jax.experimental.pallas import tpu_sc as plsc

_RESOLUTIONS = (16, 32, 64, 128)
_TABLE_SIZES = (512, 2048, 8196, 32768)
_P1 = np.int32(np.uint32(2654435761))
_P2 = np.int32(np.uint32(805459861))

_NW = 32
_LANES = 16


def _mod8196(h):
    low2 = jnp.bitwise_and(h, 3)
    h2 = lax.shift_right_logical(h, 2)
    a = lax.shift_right_logical(h2, 11)
    b = jnp.bitwise_and(h2, 2047)
    r1 = b - a + np.int32(256 * 2049)
    a2 = lax.shift_right_logical(r1, 11)
    b2 = jnp.bitwise_and(r1, 2047)
    r2 = b2 - a2 + np.int32(2049)
    t = r2 - np.int32(2049)
    r = t + jnp.bitwise_and(np.int32(2049), lax.shift_right_arithmetic(t, 31))
    return jnp.bitwise_or(lax.shift_left(r, 2), low2)


def _layout(n):
    chunk = 2000 if n % 2000 == 0 else 16
    assert n % chunk == 0
    return chunk, n // chunk


def _body(x0_hbm, x1_hbm, x2_hbm, t0l, t0h, t1l, t1h, t2l, t2h, t3l, t3h,
          out_hbm, mask_hbm,
          v0l, v0h, v1l, v1h, v2l, v2h, v3l, v3h,
          x0_v, x1_v, x2_v, out_v, mask_v, sem, *, chunk, nchunks):
    wid = lax.axis_index("s") * 2 + lax.axis_index("c")
    base_chunks = nchunks // _NW
    n_mine = base_chunks + jnp.where(wid < nchunks - base_chunks * _NW, 1, 0)
    pltpu.sync_copy(t0l, v0l)
    pltpu.sync_copy(t0h, v0h)
    pltpu.sync_copy(t1l, v1l)
    pltpu.sync_copy(t1h, v1h)
    pltpu.sync_copy(t2l, v2l)
    pltpu.sync_copy(t2h, v2h)
    pltpu.sync_copy(t3l, v3l)
    pltpu.sync_copy(t3h, v3h)

    tbls = ((v0l, v0h), (v1l, v1h), (v2l, v2h), (v3l, v3h))

    def group_body(g, _):
        lanebase = g * _LANES
        x0 = x0_v[pl.ds(lanebase, _LANES)]
        x1 = x1_v[pl.ds(lanebase, _LANES)]
        x2 = x2_v[pl.ds(lanebase, _LANES)]
        m = ((x0 >= 0.0) & (x0 <= 1.0) & (x1 >= 0.0) & (x1 <= 1.0)
             & (x2 >= 0.0) & (x2 <= 1.0))
        xc = (jnp.minimum(jnp.maximum(x0, 0.0), 1.0),
              jnp.minimum(jnp.maximum(x1, 0.0), 1.0),
              jnp.minimum(jnp.maximum(x2, 0.0), 1.0))
        for lvl in range(4):
            res = _RESOLUTIONS[lvl]
            tsize = _TABLE_SIZES[lvl]
            tlo, thi = tbls[lvl]
            scale = np.float32(res - 1)
            p0 = xc[0] * scale
            p1 = xc[1] * scale
            p2 = xc[2] * scale
            b0 = jnp.minimum(p0.astype(jnp.int32), np.int32(res - 2))
            b1 = jnp.minimum(p1.astype(jnp.int32), np.int32(res - 2))
            b2 = jnp.minimum(p2.astype(jnp.int32), np.int32(res - 2))
            f0 = p0 - b0.astype(jnp.float32)
            f1 = p1 - b1.astype(jnp.float32)
            f2 = p2 - b2.astype(jnp.float32)
            hx = (b0, b0 + 1)
            hy0 = b1 * _P1
            hy = (hy0, hy0 + _P1)
            hz0 = b2 * _P2
            hz = (hz0, hz0 + _P2)
            wx = (1.0 - f0, f0)
            wy = (1.0 - f1, f1)
            wz = (1.0 - f2, f2)
            acc0 = jnp.zeros((_LANES,), jnp.float32)
            acc1 = jnp.zeros((_LANES,), jnp.float32)
            for dx in (0, 1):
                for dy in (0, 1):
                    hxy = jnp.bitwise_xor(hx[dx], hy[dy])
                    wxy = wx[dx] * wy[dy]
                    for dz in (0, 1):
                        h = jnp.bitwise_xor(hxy, hz[dz])
                        if tsize == 8196:
                            idx = _mod8196(h)
                        else:
                            idx = jnp.bitwise_and(h, np.int32(tsize - 1))
                        w = wxy * wz[dz]
                        acc0 = acc0 + w * plsc.load_gather(tlo, [idx])
                        acc1 = acc1 + w * plsc.load_gather(thi, [idx])
            acc0 = jnp.where(m, acc0, 0.0)
            acc1 = jnp.where(m, acc1, 0.0)
            out_v[pl.ds(2 * lvl * chunk + lanebase, _LANES)] = acc0
            out_v[pl.ds((2 * lvl + 1) * chunk + lanebase, _LANES)] = acc1
        mask_v[pl.ds(lanebase, _LANES)] = jnp.where(m, 1, 0).astype(jnp.int32)
        return 0

    def chunk_body(ci, _):
        cid = ci * _NW + wid
        row0 = cid * chunk
        c0 = pltpu.async_copy(x0_hbm.at[pl.ds(row0, chunk)], x0_v, sem)
        c1 = pltpu.async_copy(x1_hbm.at[pl.ds(row0, chunk)], x1_v, sem)
        c2 = pltpu.async_copy(x2_hbm.at[pl.ds(row0, chunk)], x2_v, sem)
        c0.wait()
        c1.wait()
        c2.wait()
        lax.fori_loop(0, chunk // _LANES, group_body, 0, unroll=False)
        pltpu.sync_copy(out_v, out_hbm.at[pl.ds(cid * chunk * 8, chunk * 8)])
        pltpu.sync_copy(mask_v, mask_hbm.at[pl.ds(row0, chunk)])
        return 0

    lax.fori_loop(0, n_mine, chunk_body, 0, unroll=False)


def _build(n_pts, interpret=False):
    chunk, nchunks = _layout(n_pts)
    mesh = plsc.VectorSubcoreMesh(core_axis_name="c", subcore_axis_name="s",
                                  num_cores=2, num_subcores=16)
    scratch = [pltpu.VMEM((ts,), jnp.float32)
               for ts in _TABLE_SIZES for _ in range(2)]
    scratch += [
        pltpu.VMEM((chunk,), jnp.float32),
        pltpu.VMEM((chunk,), jnp.float32),
        pltpu.VMEM((chunk,), jnp.float32),
        pltpu.VMEM((chunk * 8,), jnp.float32),
        pltpu.VMEM((chunk,), jnp.int32),
        pltpu.SemaphoreType.DMA,
    ]
    return pl.kernel(
        functools.partial(_body, chunk=chunk, nchunks=nchunks),
        out_type=[
            jax.ShapeDtypeStruct((n_pts * 8,), jnp.float32),
            jax.ShapeDtypeStruct((n_pts,), jnp.int32),
        ],
        mesh=mesh,
        scratch_types=scratch,
        compiler_params=pltpu.CompilerParams(needs_layout_passes=False),
        interpret=interpret,
    )


@jax.jit
def kernel(x, table0, table1, table2, table3):
    n = x.shape[0]
    chunk, nchunks = _layout(n)
    k = _build(n)
    planes = []
    for t in (table0, table1, table2, table3):
        planes.append(t[:, 0])
        planes.append(t[:, 1])
    feats, mask_i32 = k(x[:, 0], x[:, 1], x[:, 2], *planes)
    feats = feats.reshape(nchunks, 8, chunk).transpose(0, 2, 1).reshape(n, 8)
    return feats, mask_i32 != 0


# speedup vs baseline: 51.9136x; 1.3139x over previous
import functools
import math

import jax
import jax.numpy as jnp
import numpy as np
from jax import lax
from jax.experimental import pallas as pl
from jax.experimental.pallas import tpu as pltpu
from jax.experimental.pallas import tpu_sc as plsc

_RESOLUTIONS = (16, 32, 64, 128)
_TABLE_SIZES = (512, 2048, 8196, 32768)
_P1 = np.int32(np.uint32(2654435761))
_P2 = np.int32(np.uint32(805459861))

_NW = 32
_LANES = 16


def _mod8196(h):
    low2 = jnp.bitwise_and(h, 3)
    h2 = lax.shift_right_logical(h, 2)
    a = lax.shift_right_logical(h2, 11)
    b = jnp.bitwise_and(h2, 2047)
    r1 = b - a + np.int32(256 * 2049)
    a2 = lax.shift_right_logical(r1, 11)
    b2 = jnp.bitwise_and(r1, 2047)
    r2 = b2 - a2 + np.int32(2049)
    t = r2 - np.int32(2049)
    r = t + jnp.bitwise_and(np.int32(2049), lax.shift_right_arithmetic(t, 31))
    return jnp.bitwise_or(lax.shift_left(r, 2), low2)


def _layout(n):
    chunk = 2000 if n % 2000 == 0 else 16
    assert n % chunk == 0
    return chunk, n // chunk


def _body(x0_hbm, x1_hbm, x2_hbm, t0l, t0h, t1l, t1h, t2l, t2h, t3l, t3h,
          out_hbm, mask_hbm,
          v0l, v0h, v1l, v1h, v2l, v2h, v3l, v3h,
          x0_v, x1_v, x2_v, out_v, mask_v, sem, *, chunk, nchunks, n_pts):
    wid = lax.axis_index("s") * 2 + lax.axis_index("c")
    base_chunks = nchunks // _NW
    n_mine = base_chunks + jnp.where(wid < nchunks - base_chunks * _NW, 1, 0)
    pltpu.sync_copy(t0l, v0l)
    pltpu.sync_copy(t0h, v0h)
    pltpu.sync_copy(t1l, v1l)
    pltpu.sync_copy(t1h, v1h)
    pltpu.sync_copy(t2l, v2l)
    pltpu.sync_copy(t2h, v2h)
    pltpu.sync_copy(t3l, v3l)
    pltpu.sync_copy(t3h, v3h)

    tbls = ((v0l, v0h), (v1l, v1h), (v2l, v2h), (v3l, v3h))

    def group_body(g, _):
        lanebase = g * _LANES
        x0 = x0_v[pl.ds(lanebase, _LANES)]
        x1 = x1_v[pl.ds(lanebase, _LANES)]
        x2 = x2_v[pl.ds(lanebase, _LANES)]
        m = ((x0 >= 0.0) & (x0 <= 1.0) & (x1 >= 0.0) & (x1 <= 1.0)
             & (x2 >= 0.0) & (x2 <= 1.0))
        xc = (jnp.minimum(jnp.maximum(x0, 0.0), 1.0),
              jnp.minimum(jnp.maximum(x1, 0.0), 1.0),
              jnp.minimum(jnp.maximum(x2, 0.0), 1.0))
        for lvl in range(4):
            res = _RESOLUTIONS[lvl]
            tsize = _TABLE_SIZES[lvl]
            tlo, thi = tbls[lvl]
            scale = np.float32(res - 1)
            p0 = xc[0] * scale
            p1 = xc[1] * scale
            p2 = xc[2] * scale
            b0 = jnp.minimum(p0.astype(jnp.int32), np.int32(res - 2))
            b1 = jnp.minimum(p1.astype(jnp.int32), np.int32(res - 2))
            b2 = jnp.minimum(p2.astype(jnp.int32), np.int32(res - 2))
            f0 = p0 - b0.astype(jnp.float32)
            f1 = p1 - b1.astype(jnp.float32)
            f2 = p2 - b2.astype(jnp.float32)
            hx = (b0, b0 + 1)
            hy0 = b1 * _P1
            hy = (hy0, hy0 + _P1)
            hz0 = b2 * _P2
            hz = (hz0, hz0 + _P2)
            wx = (1.0 - f0, f0)
            wy = (1.0 - f1, f1)
            wz = (1.0 - f2, f2)
            acc0 = jnp.zeros((_LANES,), jnp.float32)
            acc1 = jnp.zeros((_LANES,), jnp.float32)
            for dx in (0, 1):
                for dy in (0, 1):
                    hxy = jnp.bitwise_xor(hx[dx], hy[dy])
                    wxy = wx[dx] * wy[dy]
                    for dz in (0, 1):
                        h = jnp.bitwise_xor(hxy, hz[dz])
                        if tsize == 8196:
                            idx = _mod8196(h)
                        else:
                            idx = jnp.bitwise_and(h, np.int32(tsize - 1))
                        w = wxy * wz[dz]
                        acc0 = acc0 + w * plsc.load_gather(tlo, [idx])
                        acc1 = acc1 + w * plsc.load_gather(thi, [idx])
            acc0 = jnp.where(m, acc0, 0.0)
            acc1 = jnp.where(m, acc1, 0.0)
            out_v[pl.ds(2 * lvl * chunk + lanebase, _LANES)] = acc0
            out_v[pl.ds((2 * lvl + 1) * chunk + lanebase, _LANES)] = acc1
        mask_v[pl.ds(lanebase, _LANES)] = jnp.where(m, 1, 0).astype(jnp.int32)
        return 0

    def chunk_body(ci, _):
        cid = ci * _NW + wid
        row0 = cid * chunk
        c0 = pltpu.async_copy(x0_hbm.at[pl.ds(row0, chunk)], x0_v, sem)
        c1 = pltpu.async_copy(x1_hbm.at[pl.ds(row0, chunk)], x1_v, sem)
        c2 = pltpu.async_copy(x2_hbm.at[pl.ds(row0, chunk)], x2_v, sem)
        c0.wait()
        c1.wait()
        c2.wait()
        lax.fori_loop(0, chunk // _LANES, group_body, 0, unroll=False)
        outs = [
            pltpu.async_copy(
                out_v.at[pl.ds(d * chunk, chunk)],
                out_hbm.at[pl.ds(d * n_pts + row0, chunk)], sem)
            for d in range(8)
        ]
        outs.append(pltpu.async_copy(mask_v, mask_hbm.at[pl.ds(row0, chunk)],
                                     sem))
        for c in outs:
            c.wait()
        return 0

    lax.fori_loop(0, n_mine, chunk_body, 0, unroll=False)


def _build(n_pts, interpret=False):
    chunk, nchunks = _layout(n_pts)
    mesh = plsc.VectorSubcoreMesh(core_axis_name="c", subcore_axis_name="s",
                                  num_cores=2, num_subcores=16)
    scratch = [pltpu.VMEM((ts,), jnp.float32)
               for ts in _TABLE_SIZES for _ in range(2)]
    scratch += [
        pltpu.VMEM((chunk,), jnp.float32),
        pltpu.VMEM((chunk,), jnp.float32),
        pltpu.VMEM((chunk,), jnp.float32),
        pltpu.VMEM((chunk * 8,), jnp.float32),
        pltpu.VMEM((chunk,), jnp.int32),
        pltpu.SemaphoreType.DMA,
    ]
    return pl.kernel(
        functools.partial(_body, chunk=chunk, nchunks=nchunks, n_pts=n_pts),
        out_type=[
            jax.ShapeDtypeStruct((n_pts * 8,), jnp.float32),
            jax.ShapeDtypeStruct((n_pts,), jnp.int32),
        ],
        mesh=mesh,
        scratch_types=scratch,
        compiler_params=pltpu.CompilerParams(needs_layout_passes=False),
        interpret=interpret,
    )


@jax.jit
def kernel(x, table0, table1, table2, table3):
    n = x.shape[0]
    chunk, nchunks = _layout(n)
    k = _build(n)
    planes = []
    for t in (table0, table1, table2, table3):
        planes.append(t[:, 0])
        planes.append(t[:, 1])
    feats, mask_i32 = k(x[:, 0], x[:, 1], x[:, 2], *planes)
    feats = feats.reshape(8, n).T
    return feats, mask_i32 != 0


# speedup vs baseline: 98.4792x; 1.8970x over previous
import functools
import math

import jax
import jax.numpy as jnp
import numpy as np
from jax import lax
from jax.experimental import pallas as pl
from jax.experimental.pallas import tpu as pltpu
from jax.experimental.pallas import tpu_sc as plsc

_RESOLUTIONS = (16, 32, 64, 128)
_TABLE_SIZES = (512, 2048, 8196, 32768)
_P1 = np.int32(np.uint32(2654435761))
_P2 = np.int32(np.uint32(805459861))

_NW = 32
_LANES = 16


def _mod8196(h):
    low2 = jnp.bitwise_and(h, 3)
    h2 = lax.shift_right_logical(h, 2)
    a = lax.shift_right_logical(h2, 11)
    b = jnp.bitwise_and(h2, 2047)
    r1 = b - a + np.int32(256 * 2049)
    a2 = lax.shift_right_logical(r1, 11)
    b2 = jnp.bitwise_and(r1, 2047)
    r2 = b2 - a2 + np.int32(2049)
    t = r2 - np.int32(2049)
    r = t + jnp.bitwise_and(np.int32(2049), lax.shift_right_arithmetic(t, 31))
    return jnp.bitwise_or(lax.shift_left(r, 2), low2)


def _layout(n):
    chunk = 2000 if n % 2000 == 0 else 16
    assert n % chunk == 0
    return chunk, n // chunk


def _body(x0_hbm, x1_hbm, x2_hbm, t0l, t0h, t1l, t1h, t2l, t2h, t3l, t3h,
          out_hbm, mask_hbm,
          v0l, v0h, v1l, v1h, v2l, v2h, v3l, v3h,
          x0_v, x1_v, x2_v, out_v, mask_v, sem, *, chunk, nchunks, n_pts):
    wid = lax.axis_index("s") * 2 + lax.axis_index("c")
    base_chunks = nchunks // _NW
    n_mine = base_chunks + jnp.where(wid < nchunks - base_chunks * _NW, 1, 0)
    pltpu.sync_copy(t0l, v0l)
    pltpu.sync_copy(t0h, v0h)
    pltpu.sync_copy(t1l, v1l)
    pltpu.sync_copy(t1h, v1h)
    pltpu.sync_copy(t2l, v2l)
    pltpu.sync_copy(t2h, v2h)
    pltpu.sync_copy(t3l, v3l)
    pltpu.sync_copy(t3h, v3h)

    tbls = ((v0l, v0h), (v1l, v1h), (v2l, v2h), (v3l, v3h))

    def group_body(g, _):
        lanebase = g * _LANES
        x0 = x0_v[pl.ds(lanebase, _LANES)]
        x1 = x1_v[pl.ds(lanebase, _LANES)]
        x2 = x2_v[pl.ds(lanebase, _LANES)]
        m = ((x0 >= 0.0) & (x0 <= 1.0) & (x1 >= 0.0) & (x1 <= 1.0)
             & (x2 >= 0.0) & (x2 <= 1.0))
        xc = (jnp.minimum(jnp.maximum(x0, 0.0), 1.0),
              jnp.minimum(jnp.maximum(x1, 0.0), 1.0),
              jnp.minimum(jnp.maximum(x2, 0.0), 1.0))
        for lvl in range(4):
            res = _RESOLUTIONS[lvl]
            tsize = _TABLE_SIZES[lvl]
            tlo, thi = tbls[lvl]
            scale = np.float32(res - 1)
            p0 = xc[0] * scale
            p1 = xc[1] * scale
            p2 = xc[2] * scale
            b0 = jnp.minimum(p0.astype(jnp.int32), np.int32(res - 2))
            b1 = jnp.minimum(p1.astype(jnp.int32), np.int32(res - 2))
            b2 = jnp.minimum(p2.astype(jnp.int32), np.int32(res - 2))
            f0 = p0 - b0.astype(jnp.float32)
            f1 = p1 - b1.astype(jnp.float32)
            f2 = p2 - b2.astype(jnp.float32)
            hx = (b0, b0 + 1)
            hy0 = b1 * _P1
            hy = (hy0, hy0 + _P1)
            hz0 = b2 * _P2
            hz = (hz0, hz0 + _P2)
            wx = (1.0 - f0, f0)
            wy = (1.0 - f1, f1)
            wz = (1.0 - f2, f2)
            acc0 = jnp.zeros((_LANES,), jnp.float32)
            acc1 = jnp.zeros((_LANES,), jnp.float32)
            for dx in (0, 1):
                for dy in (0, 1):
                    hxy = jnp.bitwise_xor(hx[dx], hy[dy])
                    wxy = wx[dx] * wy[dy]
                    for dz in (0, 1):
                        h = jnp.bitwise_xor(hxy, hz[dz])
                        if tsize == 8196:
                            idx = _mod8196(h)
                        else:
                            idx = jnp.bitwise_and(h, np.int32(tsize - 1))
                        w = wxy * wz[dz]
                        acc0 = acc0 + w * plsc.load_gather(tlo, [idx])
                        acc1 = acc1 + w * plsc.load_gather(thi, [idx])
            acc0 = jnp.where(m, acc0, 0.0)
            acc1 = jnp.where(m, acc1, 0.0)
            out_v[pl.ds(2 * lvl * chunk + lanebase, _LANES)] = acc0
            out_v[pl.ds((2 * lvl + 1) * chunk + lanebase, _LANES)] = acc1
        mask_v[pl.ds(lanebase, _LANES)] = jnp.where(m, 1, 0).astype(jnp.int32)
        return 0

    def chunk_body(ci, _):
        cid = ci * _NW + wid
        row0 = cid * chunk
        c0 = pltpu.async_copy(x0_hbm.at[pl.ds(row0, chunk)], x0_v, sem)
        c1 = pltpu.async_copy(x1_hbm.at[pl.ds(row0, chunk)], x1_v, sem)
        c2 = pltpu.async_copy(x2_hbm.at[pl.ds(row0, chunk)], x2_v, sem)
        c0.wait()
        c1.wait()
        c2.wait()
        lax.fori_loop(0, chunk // _LANES, group_body, 0, unroll=False)
        outs = [
            pltpu.async_copy(
                out_v.at[pl.ds(d * chunk, chunk)],
                out_hbm.at[pl.ds(d * n_pts + row0, chunk)], sem)
            for d in range(8)
        ]
        outs.append(pltpu.async_copy(mask_v, mask_hbm.at[pl.ds(row0, chunk)],
                                     sem))
        for c in outs:
            c.wait()
        return 0

    lax.fori_loop(0, n_mine, chunk_body, 0, unroll=False)


def _build(n_pts, interpret=False):
    chunk, nchunks = _layout(n_pts)
    mesh = plsc.VectorSubcoreMesh(core_axis_name="c", subcore_axis_name="s",
                                  num_cores=2, num_subcores=16)
    scratch = [pltpu.VMEM((ts,), jnp.float32)
               for ts in _TABLE_SIZES for _ in range(2)]
    scratch += [
        pltpu.VMEM((chunk,), jnp.float32),
        pltpu.VMEM((chunk,), jnp.float32),
        pltpu.VMEM((chunk,), jnp.float32),
        pltpu.VMEM((chunk * 8,), jnp.float32),
        pltpu.VMEM((chunk,), jnp.int32),
        pltpu.SemaphoreType.DMA,
    ]
    return pl.kernel(
        functools.partial(_body, chunk=chunk, nchunks=nchunks, n_pts=n_pts),
        out_type=[
            jax.ShapeDtypeStruct((n_pts * 8,), jnp.float32),
            jax.ShapeDtypeStruct((n_pts,), jnp.int32),
        ],
        mesh=mesh,
        scratch_types=scratch,
        compiler_params=pltpu.CompilerParams(needs_layout_passes=False),
        interpret=interpret,
    )


@jax.jit
def kernel(x, table0, table1, table2, table3):
    n = x.shape[0]
    chunk, nchunks = _layout(n)
    k = _build(n)
    planes = []
    for t in (table0, table1, table2, table3):
        planes.append(t[:, 0])
        planes.append(t[:, 1])
    feats, mask_i32 = k(x[:, 0], x[:, 1], x[:, 2], *planes)
    feats = jnp.stack([feats[d * n:(d + 1) * n] for d in range(8)], axis=1)
    return feats, mask_i32 != 0


# speedup vs baseline: 99.5753x; 1.0111x over previous
import functools
import math

import jax
import jax.numpy as jnp
import numpy as np
from jax import lax
from jax.experimental import pallas as pl
from jax.experimental.pallas import tpu as pltpu
from jax.experimental.pallas import tpu_sc as plsc

_RESOLUTIONS = (16, 32, 64, 128)
_TABLE_SIZES = (512, 2048, 8196, 32768)
_P1 = np.int32(np.uint32(2654435761))
_P2 = np.int32(np.uint32(805459861))

_NW = 32
_LANES = 16


def _mod8196(h):
    low2 = jnp.bitwise_and(h, 3)
    h2 = lax.shift_right_logical(h, 2)
    a = lax.shift_right_logical(h2, 11)
    b = jnp.bitwise_and(h2, 2047)
    r1 = b - a + np.int32(256 * 2049)
    a2 = lax.shift_right_logical(r1, 11)
    b2 = jnp.bitwise_and(r1, 2047)
    r2 = b2 - a2 + np.int32(2049)
    t = r2 - np.int32(2049)
    r = t + jnp.bitwise_and(np.int32(2049), lax.shift_right_arithmetic(t, 31))
    return jnp.bitwise_or(lax.shift_left(r, 2), low2)


def _layout(n):
    chunk = 2000 if n % 2000 == 0 else 16
    assert n % chunk == 0
    return chunk, n // chunk


def _body(x0_hbm, x1_hbm, x2_hbm, t0l, t0h, t1l, t1h, t2l, t2h, t3l, t3h,
          out_hbm,
          v0l, v0h, v1l, v1h, v2l, v2h, v3l, v3h,
          x0_v, x1_v, x2_v, out_v, sem, *, chunk, nchunks, n_pts):
    wid = lax.axis_index("s") * 2 + lax.axis_index("c")
    base_chunks = nchunks // _NW
    n_mine = base_chunks + jnp.where(wid < nchunks - base_chunks * _NW, 1, 0)
    pltpu.sync_copy(t0l, v0l)
    pltpu.sync_copy(t0h, v0h)
    pltpu.sync_copy(t1l, v1l)
    pltpu.sync_copy(t1h, v1h)
    pltpu.sync_copy(t2l, v2l)
    pltpu.sync_copy(t2h, v2h)
    pltpu.sync_copy(t3l, v3l)
    pltpu.sync_copy(t3h, v3h)

    tbls = ((v0l, v0h), (v1l, v1h), (v2l, v2h), (v3l, v3h))

    def group_body(g, _):
        lanebase = g * _LANES
        x0 = x0_v[pl.ds(lanebase, _LANES)]
        x1 = x1_v[pl.ds(lanebase, _LANES)]
        x2 = x2_v[pl.ds(lanebase, _LANES)]
        m = ((x0 >= 0.0) & (x0 <= 1.0) & (x1 >= 0.0) & (x1 <= 1.0)
             & (x2 >= 0.0) & (x2 <= 1.0))
        xc = (jnp.minimum(jnp.maximum(x0, 0.0), 1.0),
              jnp.minimum(jnp.maximum(x1, 0.0), 1.0),
              jnp.minimum(jnp.maximum(x2, 0.0), 1.0))
        for lvl in range(4):
            res = _RESOLUTIONS[lvl]
            tsize = _TABLE_SIZES[lvl]
            tlo, thi = tbls[lvl]
            scale = np.float32(res - 1)
            p0 = xc[0] * scale
            p1 = xc[1] * scale
            p2 = xc[2] * scale
            b0 = jnp.minimum(p0.astype(jnp.int32), np.int32(res - 2))
            b1 = jnp.minimum(p1.astype(jnp.int32), np.int32(res - 2))
            b2 = jnp.minimum(p2.astype(jnp.int32), np.int32(res - 2))
            f0 = p0 - b0.astype(jnp.float32)
            f1 = p1 - b1.astype(jnp.float32)
            f2 = p2 - b2.astype(jnp.float32)
            hx = (b0, b0 + 1)
            hy0 = b1 * _P1
            hy = (hy0, hy0 + _P1)
            hz0 = b2 * _P2
            hz = (hz0, hz0 + _P2)
            wx = (1.0 - f0, f0)
            wy = (1.0 - f1, f1)
            wz = (1.0 - f2, f2)
            acc0 = jnp.zeros((_LANES,), jnp.float32)
            acc1 = jnp.zeros((_LANES,), jnp.float32)
            for dx in (0, 1):
                for dy in (0, 1):
                    hxy = jnp.bitwise_xor(hx[dx], hy[dy])
                    wxy = wx[dx] * wy[dy]
                    for dz in (0, 1):
                        h = jnp.bitwise_xor(hxy, hz[dz])
                        if tsize == 8196:
                            idx = _mod8196(h)
                        else:
                            idx = jnp.bitwise_and(h, np.int32(tsize - 1))
                        w = wxy * wz[dz]
                        acc0 = acc0 + w * plsc.load_gather(tlo, [idx])
                        acc1 = acc1 + w * plsc.load_gather(thi, [idx])
            acc0 = jnp.where(m, acc0, 0.0)
            acc1 = jnp.where(m, acc1, 0.0)
            out_v[pl.ds(2 * lvl * chunk + lanebase, _LANES)] = acc0
            out_v[pl.ds((2 * lvl + 1) * chunk + lanebase, _LANES)] = acc1
        return 0

    def chunk_body(ci, _):
        cid = ci * _NW + wid
        row0 = cid * chunk
        c0 = pltpu.async_copy(x0_hbm.at[pl.ds(row0, chunk)], x0_v, sem)
        c1 = pltpu.async_copy(x1_hbm.at[pl.ds(row0, chunk)], x1_v, sem)
        c2 = pltpu.async_copy(x2_hbm.at[pl.ds(row0, chunk)], x2_v, sem)
        c0.wait()
        c1.wait()
        c2.wait()
        lax.fori_loop(0, chunk // _LANES, group_body, 0, unroll=False)
        outs = [
            pltpu.async_copy(
                out_v.at[pl.ds(d * chunk, chunk)],
                out_hbm.at[pl.ds(d * n_pts + row0, chunk)], sem)
            for d in range(8)
        ]
        for c in outs:
            c.wait()
        return 0

    lax.fori_loop(0, n_mine, chunk_body, 0, unroll=False)


def _build(n_pts, interpret=False):
    chunk, nchunks = _layout(n_pts)
    mesh = plsc.VectorSubcoreMesh(core_axis_name="c", subcore_axis_name="s",
                                  num_cores=2, num_subcores=16)
    scratch = [pltpu.VMEM((ts,), jnp.float32)
               for ts in _TABLE_SIZES for _ in range(2)]
    scratch += [
        pltpu.VMEM((chunk,), jnp.float32),
        pltpu.VMEM((chunk,), jnp.float32),
        pltpu.VMEM((chunk,), jnp.float32),
        pltpu.VMEM((chunk * 8,), jnp.float32),
        pltpu.SemaphoreType.DMA,
    ]
    return pl.kernel(
        functools.partial(_body, chunk=chunk, nchunks=nchunks, n_pts=n_pts),
        out_type=jax.ShapeDtypeStruct((n_pts * 8,), jnp.float32),
        mesh=mesh,
        scratch_types=scratch,
        compiler_params=pltpu.CompilerParams(needs_layout_passes=False),
        interpret=interpret,
    )


@jax.jit
def kernel(x, table0, table1, table2, table3):
    n = x.shape[0]
    chunk, nchunks = _layout(n)
    k = _build(n)
    planes = []
    for t in (table0, table1, table2, table3):
        planes.append(t[:, 0])
        planes.append(t[:, 1])
    feats = k(x[:, 0], x[:, 1], x[:, 2], *planes)
    feats = jnp.stack([feats[d * n:(d + 1) * n] for d in range(8)], axis=1)
    mask = jnp.all((x >= 0.0) & (x <= 1.0), axis=-1)
    return feats, mask


# speedup vs baseline: 121.7018x; 1.2222x over previous
import functools
import math

import jax
import jax.numpy as jnp
import numpy as np
from jax import lax
from jax.experimental import pallas as pl
from jax.experimental.pallas import tpu as pltpu
from jax.experimental.pallas import tpu_sc as plsc

_RESOLUTIONS = (16, 32, 64, 128)
_TABLE_SIZES = (512, 2048, 8196, 32768)
_P1 = np.int32(np.uint32(2654435761))
_P2 = np.int32(np.uint32(805459861))

_NW = 32
_LANES = 16


def _mod8196(h):
    low2 = jnp.bitwise_and(h, 3)
    h2 = lax.shift_right_logical(h, 2)
    a = lax.shift_right_logical(h2, 11)
    b = jnp.bitwise_and(h2, 2047)
    r1 = b - a + np.int32(256 * 2049)
    a2 = lax.shift_right_logical(r1, 11)
    b2 = jnp.bitwise_and(r1, 2047)
    r2 = b2 - a2 + np.int32(2049)
    t = r2 - np.int32(2049)
    r = t + jnp.bitwise_and(np.int32(2049), lax.shift_right_arithmetic(t, 31))
    return jnp.bitwise_or(lax.shift_left(r, 2), low2)


def _layout(n):
    chunk = 2000 if n % 2000 == 0 else 16
    assert n % chunk == 0
    return chunk, n // chunk


def _body(x0_hbm, x1_hbm, x2_hbm, t0l, t0h, t1l, t1h, t2l, t2h, t3l, t3h,
          out_hbm,
          v0l, v0h, v1l, v1h, v2l, v2h, v3l, v3h,
          x0_v, x1_v, x2_v, out_v, sem, *, chunk, nchunks, n_pts):
    wid = lax.axis_index("s") * 2 + lax.axis_index("c")
    base_chunks = nchunks // _NW
    n_mine = base_chunks + jnp.where(wid < nchunks - base_chunks * _NW, 1, 0)
    pltpu.sync_copy(t0l, v0l)
    pltpu.sync_copy(t0h, v0h)
    pltpu.sync_copy(t1l, v1l)
    pltpu.sync_copy(t1h, v1h)
    pltpu.sync_copy(t2l, v2l)
    pltpu.sync_copy(t2h, v2h)
    pltpu.sync_copy(t3l, v3l)
    pltpu.sync_copy(t3h, v3h)

    tbls = ((v0l, v0h), (v1l, v1h), (v2l, v2h), (v3l, v3h))

    def group_body(g, _):
        lanebase = g * _LANES
        x0 = x0_v[pl.ds(lanebase, _LANES)]
        x1 = x1_v[pl.ds(lanebase, _LANES)]
        x2 = x2_v[pl.ds(lanebase, _LANES)]
        m = ((x0 >= 0.0) & (x0 <= 1.0) & (x1 >= 0.0) & (x1 <= 1.0)
             & (x2 >= 0.0) & (x2 <= 1.0))
        xc = (jnp.minimum(jnp.maximum(x0, 0.0), 1.0),
              jnp.minimum(jnp.maximum(x1, 0.0), 1.0),
              jnp.minimum(jnp.maximum(x2, 0.0), 1.0))
        for lvl in range(4):
            res = _RESOLUTIONS[lvl]
            tsize = _TABLE_SIZES[lvl]
            tlo, thi = tbls[lvl]
            scale = np.float32(res - 1)
            p0 = xc[0] * scale
            p1 = xc[1] * scale
            p2 = xc[2] * scale
            b0 = jnp.minimum(p0.astype(jnp.int32), np.int32(res - 2))
            b1 = jnp.minimum(p1.astype(jnp.int32), np.int32(res - 2))
            b2 = jnp.minimum(p2.astype(jnp.int32), np.int32(res - 2))
            f0 = p0 - b0.astype(jnp.float32)
            f1 = p1 - b1.astype(jnp.float32)
            f2 = p2 - b2.astype(jnp.float32)
            hx = (b0, b0 + 1)
            hy0 = b1 * _P1
            hy = (hy0, hy0 + _P1)
            hz0 = b2 * _P2
            hz = (hz0, hz0 + _P2)
            wx = (1.0 - f0, f0)
            wy = (1.0 - f1, f1)
            wz = (1.0 - f2, f2)
            acc0 = jnp.zeros((_LANES,), jnp.float32)
            acc1 = jnp.zeros((_LANES,), jnp.float32)
            for dx in (0, 1):
                for dy in (0, 1):
                    hxy = jnp.bitwise_xor(hx[dx], hy[dy])
                    wxy = wx[dx] * wy[dy]
                    for dz in (0, 1):
                        h = jnp.bitwise_xor(hxy, hz[dz])
                        if tsize == 8196:
                            idx = _mod8196(h)
                        else:
                            idx = jnp.bitwise_and(h, np.int32(tsize - 1))
                        w = wxy * wz[dz]
                        acc0 = acc0 + w * plsc.load_gather(tlo, [idx])
                        acc1 = acc1 + w * plsc.load_gather(thi, [idx])
            acc0 = jnp.where(m, acc0, 0.0)
            acc1 = jnp.where(m, acc1, 0.0)
            out_v[pl.ds(2 * lvl * chunk + lanebase, _LANES)] = acc0
            out_v[pl.ds((2 * lvl + 1) * chunk + lanebase, _LANES)] = acc1
        return 0

    def chunk_body(ci, _):
        cid = ci * _NW + wid
        row0 = cid * chunk
        c0 = pltpu.async_copy(x0_hbm.at[pl.ds(row0, chunk)], x0_v, sem)
        c1 = pltpu.async_copy(x1_hbm.at[pl.ds(row0, chunk)], x1_v, sem)
        c2 = pltpu.async_copy(x2_hbm.at[pl.ds(row0, chunk)], x2_v, sem)
        c0.wait()
        c1.wait()
        c2.wait()
        lax.fori_loop(0, chunk // _LANES, group_body, 0, unroll=False)
        outs = [
            pltpu.async_copy(
                out_v.at[pl.ds(d * chunk, chunk)],
                out_hbm.at[pl.ds(d * n_pts + row0, chunk)], sem)
            for d in range(8)
        ]
        for c in outs:
            c.wait()
        return 0

    lax.fori_loop(0, n_mine, chunk_body, 0, unroll=False)


def _build(n_pts, interpret=False):
    chunk, nchunks = _layout(n_pts)
    mesh = plsc.VectorSubcoreMesh(core_axis_name="c", subcore_axis_name="s",
                                  num_cores=2, num_subcores=16)
    scratch = [pltpu.VMEM((ts,), jnp.float32)
               for ts in _TABLE_SIZES for _ in range(2)]
    scratch += [
        pltpu.VMEM((chunk,), jnp.float32),
        pltpu.VMEM((chunk,), jnp.float32),
        pltpu.VMEM((chunk,), jnp.float32),
        pltpu.VMEM((chunk * 8,), jnp.float32),
        pltpu.SemaphoreType.DMA,
    ]
    return pl.kernel(
        functools.partial(_body, chunk=chunk, nchunks=nchunks, n_pts=n_pts),
        out_type=jax.ShapeDtypeStruct((n_pts * 8,), jnp.float32),
        mesh=mesh,
        scratch_types=scratch,
        compiler_params=pltpu.CompilerParams(needs_layout_passes=False),
        interpret=interpret,
    )


def _segments(n):
    if n == 1000000:
        return [(0, 256000), (256000, 256000), (512000, 256000),
                (768000, 232000)]
    return [(0, n)]


@jax.jit
def kernel(x, table0, table1, table2, table3):
    n = x.shape[0]
    x0, x1, x2 = x[:, 0], x[:, 1], x[:, 2]
    planes = []
    for t in (table0, table1, table2, table3):
        planes.append(t[:, 0])
        planes.append(t[:, 1])
    parts = []
    for s0, sn in _segments(n):
        k = _build(sn)
        f = k(lax.dynamic_slice(x0, (s0,), (sn,)),
              lax.dynamic_slice(x1, (s0,), (sn,)),
              lax.dynamic_slice(x2, (s0,), (sn,)), *planes)
        parts.append(jnp.stack([f[d * sn:(d + 1) * sn] for d in range(8)],
                               axis=1))
    feats = jnp.concatenate(parts, axis=0) if len(parts) > 1 else parts[0]
    mask = jnp.all((x >= 0.0) & (x <= 1.0), axis=-1)
    return feats, mask
